# baseline (speedup 1.0000x reference)
import functools

import jax
import jax.numpy as jnp
from jax import lax
from jax.experimental import pallas as pl
from jax.experimental.pallas import tpu as pltpu, tpu_sc as plsc


def _make(B, V, NC, NS):
    NW = NC * NS
    b_per_w = B // NW
    BATCH16 = 16
    n_batch = b_per_w // BATCH16
    mesh = plsc.VectorSubcoreMesh(core_axis_name="c", subcore_axis_name="s")

    @functools.partial(
        pl.kernel,
        mesh=mesh,
        out_type=jax.ShapeDtypeStruct((32, B), jnp.float32),
        scratch_types=[
            pltpu.VMEM((b_per_w,), jnp.int32),
            pltpu.VMEM((BATCH16, 32, 128), jnp.float32),
            pltpu.VMEM((32, b_per_w), jnp.float32),
            pltpu.SemaphoreType.DMA,
        ],
        compiler_params=pltpu.CompilerParams(
            disable_bounds_checks=True, needs_layout_passes=False
        ),
    )
    def k(idx_hbm, tab_hbm, out_hbm, idx_v, bufs, stage, sem):
        wid = lax.axis_index("s") * NC + lax.axis_index("c")
        base = wid * b_per_w
        pltpu.sync_copy(idx_hbm.at[pl.ds(base, b_per_w)], idx_v)
        row16 = lax.iota(jnp.int32, 16)

        def fire(cv, l):
            start = pl.multiple_of(cv[l] & ~jnp.int32(127), 128)
            return pltpu.async_copy(
                tab_hbm.at[:, pl.ds(start, 128)], bufs.at[l], sem
            )

        def extract(cv, j0, l):
            cl = jnp.full((16,), cv[l] & 127, jnp.int32)
            jv = jnp.full((16,), j0 + l, jnp.int32)
            v0 = plsc.load_gather(bufs.at[l], [row16, cl])
            v1 = plsc.load_gather(bufs.at[l], [row16 + 16, cl])
            plsc.store_scatter(stage, [row16, jv], v0)
            plsc.store_scatter(stage, [row16 + 16, jv], v1)

        def batch(b, _):
            j0 = b * BATCH16
            cv = idx_v[pl.ds(j0, BATCH16)]
            copies_a = [fire(cv, l) for l in range(8)]
            copies_b = [fire(cv, l) for l in range(8, 16)]
            for c in copies_a:
                c.wait()
            for l in range(8):
                extract(cv, j0, l)
            for c in copies_b:
                c.wait()
            for l in range(8, 16):
                extract(cv, j0, l)
            return _

        lax.fori_loop(0, n_batch, batch, 0)
        pltpu.sync_copy(stage, out_hbm.at[:, pl.ds(base, b_per_w)])

    return k


def kernel(class_id, table):
    (B,) = class_id.shape
    V, D = table.shape
    info = plsc.get_sparse_core_info()
    NC, NS = info.num_cores, info.num_subcores
    tt = table.T
    out_t = _make(B, V, NC, NS)(class_id.astype(jnp.int32), tt)
    return out_t.T


# speedup vs baseline: 1.0425x; 1.0425x over previous
import functools

import jax
import jax.numpy as jnp
from jax import lax
from jax.experimental import pallas as pl
from jax.experimental.pallas import tpu as pltpu, tpu_sc as plsc


def _make(B, V, NC, NS):
    NW = NC * NS
    b_per_w = B // NW
    BATCH16 = 16
    n_batch = b_per_w // BATCH16
    mesh = plsc.VectorSubcoreMesh(core_axis_name="c", subcore_axis_name="s")

    @functools.partial(
        pl.kernel,
        mesh=mesh,
        out_type=jax.ShapeDtypeStruct((32, B), jnp.float32),
        scratch_types=[
            pltpu.VMEM((b_per_w,), jnp.int32),
            pltpu.VMEM((BATCH16, 32, 128), jnp.float32),
            pltpu.VMEM((32, b_per_w), jnp.float32),
            pltpu.SemaphoreType.DMA,
        ],
        compiler_params=pltpu.CompilerParams(
            disable_bounds_checks=True, needs_layout_passes=False
        ),
    )
    def k(idx_hbm, tab_hbm, out_hbm, idx_v, bufs, stage, sem):
        wid = lax.axis_index("s") * NC + lax.axis_index("c")
        base = wid * b_per_w
        pltpu.sync_copy(idx_hbm.at[pl.ds(base, b_per_w)], idx_v)
        row16 = lax.iota(jnp.int32, 16)

        def fire(cv, l):
            start = pl.multiple_of(cv[l] & ~jnp.int32(127), 128)
            return pltpu.async_copy(
                tab_hbm.at[:, pl.ds(start, 128)], bufs.at[l], sem
            )

        def extract(cv, j0, l):
            cl = jnp.full((16,), cv[l] & 127, jnp.int32)
            jv = jnp.full((16,), j0 + l, jnp.int32)
            v0 = plsc.load_gather(bufs.at[l], [row16, cl])
            v1 = plsc.load_gather(bufs.at[l], [row16 + 16, cl])
            plsc.store_scatter(stage, [row16, jv], v0)
            plsc.store_scatter(stage, [row16 + 16, jv], v1)

        def batch(b, _):
            j0 = b * BATCH16
            cv = idx_v[pl.ds(j0, BATCH16)]
            copies = [fire(cv, l) for l in range(16)]
            for l in range(16):
                copies[l].wait()
                extract(cv, j0, l)
            return _

        lax.fori_loop(0, n_batch, batch, 0)
        pltpu.sync_copy(stage, out_hbm.at[:, pl.ds(base, b_per_w)])

    return k


def kernel(class_id, table):
    (B,) = class_id.shape
    V, D = table.shape
    info = plsc.get_sparse_core_info()
    NC, NS = info.num_cores, info.num_subcores
    tt = table.T
    out_t = _make(B, V, NC, NS)(class_id.astype(jnp.int32), tt)
    return out_t.T
